# baseline (speedup 1.0000x reference)
import functools

import jax
import jax.numpy as jnp
from jax import lax
from jax.experimental import pallas as pl
from jax.experimental.pallas import tpu as pltpu
from jax.experimental.pallas import tpu_sc as plsc

N = 10000
E = 320000
D = 128

NC = 2
NS = 16
NW = NC * NS
EPT = E // NW
CHUNK = 64
NCHP = 160
EPTP = NCHP * CHUNK
NSINK = 8
NP = N + NSINK
SLAB = 624
REM = N - NS * SLAB
ZREM = NP - NS * SLAB
ZR = 16


def _segsum_body(g_hbm, srcp_hbm, dstp_hbm, out_hbm, agg_sh, sidx, didx,
                 zbuf, rows0, rows1, rows2, gsem0, gsem1, gsem2,
                 ssem0, ssem1, ssem2, isem):
    rows = (rows0, rows1, rows2)
    gsems = (gsem0, gsem1, gsem2)
    ssems = (ssem0, ssem1, ssem2)
    c = lax.axis_index("c")
    s = lax.axis_index("s")
    wid = c * NS + s
    base = wid * EPTP

    sd = pltpu.async_copy(srcp_hbm.at[pl.ds(base, EPTP)], sidx, isem)
    dd = pltpu.async_copy(dstp_hbm.at[pl.ds(base, EPTP)], didx, isem)

    zero16 = jnp.zeros((16,), jnp.float32)

    def zfill(i, carry):
        for k in range(D // 16):
            zbuf[i, pl.ds(k * 16, 16)] = zero16
        return carry

    lax.fori_loop(0, ZR, zfill, 0)
    for q in range(SLAB // ZR):
        pltpu.sync_copy(zbuf, agg_sh.at[pl.ds(s * SLAB + q * ZR, ZR)])

    @pl.when(s == NS - 1)
    def _zero_rem():
        pltpu.sync_copy(zbuf, agg_sh.at[pl.ds(NS * SLAB, ZR)])
        pltpu.sync_copy(zbuf.at[pl.ds(0, ZREM - ZR)],
                        agg_sh.at[pl.ds(NS * SLAB + ZR, ZREM - ZR)])

    sd.wait()
    dd.wait()
    plsc.subcore_barrier()

    def fire_gather(j, b):
        pltpu.async_copy(g_hbm.at[sidx.at[pl.ds(j * CHUNK, CHUNK)]],
                         rows[b], gsems[b])

    def wait_gather(j, b):
        pltpu.make_async_copy(g_hbm.at[sidx.at[pl.ds(j * CHUNK, CHUNK)]],
                              rows[b], gsems[b]).wait()

    def fire_scatter(j, b):
        pltpu.async_copy(rows[b], agg_sh.at[didx.at[pl.ds(j * CHUNK, CHUNK)]],
                         ssems[b], add=True)

    def wait_scatter(j, b):
        pltpu.make_async_copy(rows[b],
                              agg_sh.at[didx.at[pl.ds(j * CHUNK, CHUNK)]],
                              ssems[b]).wait()

    fire_gather(0, 0)
    fire_gather(1, 1)
    wait_gather(0, 0)
    fire_scatter(0, 0)
    fire_gather(2, 2)

    def steady(g, carry):
        jb = 3 * g + 1
        for k in range(3):
            j = jb + k
            b = (1 + k) % 3
            wait_gather(j, b)
            fire_scatter(j, b)
            wait_scatter(j - 1, k % 3)
            fire_gather(j + 2, k % 3)
        return carry

    lax.fori_loop(0, 52, steady, 0)
    for j in range(157, 158):
        b = j % 3
        wait_gather(j, b)
        fire_scatter(j, b)
        wait_scatter(j - 1, (j - 1) % 3)
        fire_gather(j + 2, (j - 1) % 3)
    for j in range(158, 160):
        b = j % 3
        wait_gather(j, b)
        fire_scatter(j, b)
        wait_scatter(j - 1, (j - 1) % 3)
    wait_scatter(NCHP - 1, (NCHP - 1) % 3)
    plsc.subcore_barrier()

    pltpu.sync_copy(agg_sh.at[pl.ds(s * SLAB, SLAB)],
                    out_hbm.at[c, pl.ds(s * SLAB, SLAB)])

    @pl.when(s == NS - 1)
    def _drain_rem():
        pltpu.sync_copy(agg_sh.at[pl.ds(NS * SLAB, REM)],
                        out_hbm.at[c, pl.ds(NS * SLAB, REM)])


def _make_segsum():
    mesh = plsc.VectorSubcoreMesh(core_axis_name="c", subcore_axis_name="s")
    scratch = [
        pltpu.VMEM_SHARED((NP, D), jnp.float32),
        pltpu.VMEM((EPTP,), jnp.int32),
        pltpu.VMEM((EPTP,), jnp.int32),
        pltpu.VMEM((ZR, D), jnp.float32),
        pltpu.VMEM((CHUNK, D), jnp.float32),
        pltpu.VMEM((CHUNK, D), jnp.float32),
        pltpu.VMEM((CHUNK, D), jnp.float32),
        pltpu.SemaphoreType.DMA,
        pltpu.SemaphoreType.DMA,
        pltpu.SemaphoreType.DMA,
        pltpu.SemaphoreType.DMA,
        pltpu.SemaphoreType.DMA,
        pltpu.SemaphoreType.DMA,
        pltpu.SemaphoreType.DMA,
    ]
    return pl.kernel(
        _segsum_body,
        out_type=jax.ShapeDtypeStruct((NC, N, D), jnp.float32),
        mesh=mesh,
        scratch_types=scratch,
    )


def _mm_body(x_ref, w_ref, o_ref):
    o_ref[...] = lax.dot_general(
        x_ref[...], w_ref[...], (((1,), (1,)), ((), ())),
        preferred_element_type=jnp.float32)


def _mm(x, w):
    return pl.pallas_call(
        _mm_body,
        grid=(10,),
        in_specs=[
            pl.BlockSpec((N // 10, D), lambda i: (i, 0)),
            pl.BlockSpec((D, D), lambda i: (0, 0)),
        ],
        out_specs=pl.BlockSpec((N // 10, D), lambda i: (i, 0)),
        out_shape=jax.ShapeDtypeStruct((N, D), jnp.float32),
    )(x, w)


def _combine_mm_body(scale_ref, g_ref, agg_ref, b_ref, w_ref, o_ref):
    z = (scale_ref[0] * g_ref[...] + agg_ref[0] + agg_ref[1]
         + b_ref[...][None, :])
    o_ref[...] = lax.dot_general(
        z, w_ref[...], (((1,), (1,)), ((), ())),
        preferred_element_type=jnp.float32)


def _combine_mm(scale, g, agg, b, w):
    return pl.pallas_call(
        _combine_mm_body,
        grid=(10,),
        in_specs=[
            pl.BlockSpec(memory_space=pltpu.SMEM),
            pl.BlockSpec((N // 10, D), lambda i: (i, 0)),
            pl.BlockSpec((NC, N // 10, D), lambda i: (0, i, 0)),
            pl.BlockSpec((D,), lambda i: (0,)),
            pl.BlockSpec((D, D), lambda i: (0, 0)),
        ],
        out_specs=pl.BlockSpec((N // 10, D), lambda i: (i, 0)),
        out_shape=jax.ShapeDtypeStruct((N, D), jnp.float32),
    )(scale, g, agg, b, w)


def _combine_body(scale_ref, g_ref, agg_ref, b_ref, o_ref):
    o_ref[...] = (scale_ref[0] * g_ref[...] + agg_ref[0] + agg_ref[1]
                  + b_ref[...][None, :])


def _combine(scale, g, agg, b):
    return pl.pallas_call(
        _combine_body,
        grid=(10,),
        in_specs=[
            pl.BlockSpec(memory_space=pltpu.SMEM),
            pl.BlockSpec((N // 10, D), lambda i: (i, 0)),
            pl.BlockSpec((NC, N // 10, D), lambda i: (0, i, 0)),
            pl.BlockSpec((D,), lambda i: (0,)),
        ],
        out_specs=pl.BlockSpec((N // 10, D), lambda i: (i, 0)),
        out_shape=jax.ShapeDtypeStruct((N, D), jnp.float32),
    )(scale, g, agg, b)


_segsum = _make_segsum()


def kernel(feats, edge_index, W1, b1, W2, b2, eps1, eps2):
    npad = EPTP - EPT
    src = edge_index[0].reshape(NW, EPT)
    dst = edge_index[1].reshape(NW, EPT)
    srcp = jnp.pad(src, ((0, 0), (0, npad))).reshape(-1)
    sink = jnp.broadcast_to(
        N + (jnp.arange(npad, dtype=jnp.int32) % NSINK), (NW, npad))
    dstp = jnp.concatenate([dst, sink], axis=1).reshape(-1)
    scale1 = (1.0 + eps1).reshape(1)
    scale2 = (1.0 + eps2).reshape(1)
    g1 = _mm(feats, W1)
    agg1 = _segsum(g1, srcp, dstp)
    g2 = _combine_mm(scale1, g1, agg1, b1, W2)
    agg2 = _segsum(g2, srcp, dstp)
    return _combine(scale2, g2, agg2, b2)


# speedup vs baseline: 2.9711x; 2.9711x over previous
import functools

import jax
import jax.numpy as jnp
from jax import lax
from jax.experimental import pallas as pl
from jax.experimental.pallas import tpu as pltpu
from jax.experimental.pallas import tpu_sc as plsc

N = 10000
E = 320000
D = 128

NC = 2
NS = 16
NW = NC * NS
EPT = E // NW
CHUNK = 64
NCHP = 160
EPTP = NCHP * CHUNK
NPAD = EPTP - EPT
NZ = NPAD
NP = N
SLAB = 624
REM = N - NS * SLAB
ZR = 16


def _segsum_body(g_hbm, srcp_hbm, dstp_hbm, out_hbm, agg_sh, sidx, didx,
                 zbuf, rows0, rows1, rows2, gsem0, gsem1, gsem2,
                 ssem0, ssem1, ssem2, isem):
    rows = (rows0, rows1, rows2)
    gsems = (gsem0, gsem1, gsem2)
    ssems = (ssem0, ssem1, ssem2)
    c = lax.axis_index("c")
    s = lax.axis_index("s")
    wid = c * NS + s
    base = wid * EPTP

    sd = pltpu.async_copy(srcp_hbm.at[pl.ds(base, EPTP)], sidx, isem)
    dd = pltpu.async_copy(dstp_hbm.at[pl.ds(base, EPTP)], didx, isem)

    zero16 = jnp.zeros((16,), jnp.float32)

    def zfill(i, carry):
        for k in range(D // 16):
            zbuf[i, pl.ds(k * 16, 16)] = zero16
        return carry

    lax.fori_loop(0, ZR, zfill, 0)
    for q in range(SLAB // ZR):
        pltpu.sync_copy(zbuf, agg_sh.at[pl.ds(s * SLAB + q * ZR, ZR)])

    @pl.when(s == NS - 1)
    def _zero_rem():
        pltpu.sync_copy(zbuf, agg_sh.at[pl.ds(NS * SLAB, REM)])

    sd.wait()
    dd.wait()
    plsc.subcore_barrier()

    def fire_gather(j, b):
        pltpu.async_copy(g_hbm.at[sidx.at[pl.ds(j * CHUNK, CHUNK)]],
                         rows[b], gsems[b])

    def wait_gather(j, b):
        pltpu.make_async_copy(g_hbm.at[sidx.at[pl.ds(j * CHUNK, CHUNK)]],
                              rows[b], gsems[b]).wait()

    def fire_scatter(j, b):
        pltpu.async_copy(rows[b], agg_sh.at[didx.at[pl.ds(j * CHUNK, CHUNK)]],
                         ssems[b], add=True)

    def wait_scatter(j, b):
        pltpu.make_async_copy(rows[b],
                              agg_sh.at[didx.at[pl.ds(j * CHUNK, CHUNK)]],
                              ssems[b]).wait()

    fire_gather(0, 0)
    fire_gather(1, 1)
    wait_gather(0, 0)
    fire_scatter(0, 0)
    fire_gather(2, 2)

    def steady(g, carry):
        jb = 3 * g + 1
        for k in range(3):
            j = jb + k
            b = (1 + k) % 3
            wait_gather(j, b)
            fire_scatter(j, b)
            wait_scatter(j - 1, k % 3)
            fire_gather(j + 2, k % 3)
        return carry

    lax.fori_loop(0, 52, steady, 0)
    for j in range(157, 158):
        b = j % 3
        wait_gather(j, b)
        fire_scatter(j, b)
        wait_scatter(j - 1, (j - 1) % 3)
        fire_gather(j + 2, (j - 1) % 3)
    for j in range(158, 160):
        b = j % 3
        wait_gather(j, b)
        fire_scatter(j, b)
        wait_scatter(j - 1, (j - 1) % 3)
    wait_scatter(NCHP - 1, (NCHP - 1) % 3)
    plsc.subcore_barrier()

    pltpu.sync_copy(agg_sh.at[pl.ds(s * SLAB, SLAB)],
                    out_hbm.at[c, pl.ds(s * SLAB, SLAB)])

    @pl.when(s == NS - 1)
    def _drain_rem():
        pltpu.sync_copy(agg_sh.at[pl.ds(NS * SLAB, REM)],
                        out_hbm.at[c, pl.ds(NS * SLAB, REM)])


def _make_segsum():
    mesh = plsc.VectorSubcoreMesh(core_axis_name="c", subcore_axis_name="s")
    scratch = [
        pltpu.VMEM_SHARED((NP, D), jnp.float32),
        pltpu.VMEM((EPTP,), jnp.int32),
        pltpu.VMEM((EPTP,), jnp.int32),
        pltpu.VMEM((ZR, D), jnp.float32),
        pltpu.VMEM((CHUNK, D), jnp.float32),
        pltpu.VMEM((CHUNK, D), jnp.float32),
        pltpu.VMEM((CHUNK, D), jnp.float32),
        pltpu.SemaphoreType.DMA,
        pltpu.SemaphoreType.DMA,
        pltpu.SemaphoreType.DMA,
        pltpu.SemaphoreType.DMA,
        pltpu.SemaphoreType.DMA,
        pltpu.SemaphoreType.DMA,
        pltpu.SemaphoreType.DMA,
    ]
    return pl.kernel(
        _segsum_body,
        out_type=jax.ShapeDtypeStruct((NC, N, D), jnp.float32),
        mesh=mesh,
        scratch_types=scratch,
    )


def _mm_body(x_ref, w_ref, o_ref):
    o_ref[...] = lax.dot_general(
        x_ref[...], w_ref[...], (((1,), (1,)), ((), ())),
        preferred_element_type=jnp.float32)


def _mm(x, w):
    return pl.pallas_call(
        _mm_body,
        grid=(10,),
        in_specs=[
            pl.BlockSpec((N // 10, D), lambda i: (i, 0)),
            pl.BlockSpec((D, D), lambda i: (0, 0)),
        ],
        out_specs=pl.BlockSpec((N // 10, D), lambda i: (i, 0)),
        out_shape=jax.ShapeDtypeStruct((N, D), jnp.float32),
    )(x, w)


def _combine_mm_body(scale_ref, g_ref, agg_ref, b_ref, w_ref, o_ref):
    z = (scale_ref[0] * g_ref[...] + agg_ref[0] + agg_ref[1]
         + b_ref[...][None, :])
    o_ref[...] = lax.dot_general(
        z, w_ref[...], (((1,), (1,)), ((), ())),
        preferred_element_type=jnp.float32)


def _combine_mm(scale, g, agg, b, w):
    return pl.pallas_call(
        _combine_mm_body,
        grid=(10,),
        in_specs=[
            pl.BlockSpec(memory_space=pltpu.SMEM),
            pl.BlockSpec((N // 10, D), lambda i: (i, 0)),
            pl.BlockSpec((NC, N // 10, D), lambda i: (0, i, 0)),
            pl.BlockSpec((D,), lambda i: (0,)),
            pl.BlockSpec((D, D), lambda i: (0, 0)),
        ],
        out_specs=pl.BlockSpec((N // 10, D), lambda i: (i, 0)),
        out_shape=jax.ShapeDtypeStruct((N, D), jnp.float32),
    )(scale, g, agg, b, w)


def _combine_body(scale_ref, g_ref, agg_ref, b_ref, o_ref):
    o_ref[...] = (scale_ref[0] * g_ref[...] + agg_ref[0] + agg_ref[1]
                  + b_ref[...][None, :])


def _combine(scale, g, agg, b):
    return pl.pallas_call(
        _combine_body,
        grid=(10,),
        in_specs=[
            pl.BlockSpec(memory_space=pltpu.SMEM),
            pl.BlockSpec((N // 10, D), lambda i: (i, 0)),
            pl.BlockSpec((NC, N // 10, D), lambda i: (0, i, 0)),
            pl.BlockSpec((D,), lambda i: (0,)),
        ],
        out_specs=pl.BlockSpec((N // 10, D), lambda i: (i, 0)),
        out_shape=jax.ShapeDtypeStruct((N, D), jnp.float32),
    )(scale, g, agg, b)


_segsum = _make_segsum()


_ZROWS = None


def kernel(feats, edge_index, W1, b1, W2, b2, eps1, eps2):
    src = edge_index[0].reshape(NW, EPT)
    dst = edge_index[1].reshape(NW, EPT)
    pos = (jnp.arange(NW, dtype=jnp.int32)[:, None] * NPAD
           + jnp.arange(NPAD, dtype=jnp.int32)[None, :])
    srcp = jnp.concatenate([src, N + (pos % NZ)], axis=1).reshape(-1)
    dstp = jnp.concatenate([dst, (pos * 41) % N], axis=1).reshape(-1)
    zrows = jnp.zeros((NZ, D), jnp.float32)
    scale1 = (1.0 + eps1).reshape(1)
    scale2 = (1.0 + eps2).reshape(1)
    g1 = _mm(feats, W1)
    agg1 = _segsum(jnp.concatenate([g1, zrows]), srcp, dstp)
    g2 = _combine_mm(scale1, g1, agg1, b1, W2)
    agg2 = _segsum(jnp.concatenate([g2, zrows]), srcp, dstp)
    return _combine(scale2, g2, agg2, b2)
